# baseline (speedup 1.0000x reference)
import functools

import jax
import jax.numpy as jnp
from jax import lax
from jax.experimental import pallas as pl
from jax.experimental.pallas import tpu as pltpu
from jax.experimental.pallas import tpu_sc as plsc

VOCAB = 1000000
D = 64
B = 4096
L = 200
N = B * L

CH = 128
K = 8
SUPER = CH * K


def _make_gather():
  info = plsc.get_sparse_core_info()
  nc, ns = info.num_cores, info.num_subcores
  nw = nc * ns
  per_w = N // nw
  n_super = per_w // SUPER
  idx_rows_per_w = per_w // CH

  mesh = plsc.VectorSubcoreMesh(core_axis_name="c", subcore_axis_name="s")

  @functools.partial(
      pl.kernel,
      mesh=mesh,
      out_type=jax.ShapeDtypeStruct((N, D), jnp.float32),
      scratch_types=[
          pltpu.VMEM((K, CH), jnp.int32),
          pltpu.VMEM((SUPER, D), jnp.float32),
          pltpu.SemaphoreType.DMA,
      ],
      compiler_params=pltpu.CompilerParams(use_tc_tiling_on_sc=False),
  )
  def gather_kernel(table_hbm, idx_hbm, out_hbm, idx_v, rows_v, sem):
    wid = lax.axis_index("s") * nc + lax.axis_index("c")
    idx_row0 = wid * idx_rows_per_w
    out_row0 = wid * per_w

    def body(g, carry):
      pltpu.sync_copy(idx_hbm.at[pl.ds(idx_row0 + g * K, K)], idx_v)
      copies = []
      for j in range(K):
        copies.append(
            pltpu.async_copy(
                table_hbm.at[idx_v.at[j]],
                rows_v.at[pl.ds(j * CH, CH)],
                sem,
            )
        )
      for c in copies:
        c.wait()
      pltpu.sync_copy(rows_v, out_hbm.at[pl.ds(out_row0 + g * SUPER, SUPER)])
      return carry

    lax.fori_loop(0, n_super, body, 0)

  return gather_kernel


_gather = _make_gather()


@jax.jit
def kernel(x, embed_weight):
  idx = x.reshape(-1).astype(jnp.int32).reshape(N // CH, CH)
  out = _gather(embed_weight, idx)
  return out.reshape(B, L, D)
